# baseline (speedup 1.0000x reference)
import functools

import jax
import jax.numpy as jnp
from jax import lax
from jax.experimental import pallas as pl
from jax.experimental.pallas import tpu as pltpu
from jax.experimental.pallas import tpu_sc as plsc

N1 = 2048
N2 = 8192
B = 512
D = 128
HID = 128

_F32 = jnp.float32

_L = 16


@functools.lru_cache(maxsize=None)
def _make_sc_gather_l1():
    info = plsc.get_sparse_core_info()
    NC, NS = info.num_cores, info.num_subcores
    NW = NC * NS
    B_ = 2 * (N2 + N1)
    b_per_w = B_ // NW
    CH = 128
    n_ch = b_per_w // CH
    mesh = plsc.VectorSubcoreMesh(core_axis_name="c", subcore_axis_name="s")

    @functools.partial(
        pl.kernel,
        mesh=mesh,
        out_type=jax.ShapeDtypeStruct((B_, D), _F32),
        scratch_types=[
            pltpu.VMEM((n_ch, CH), jnp.int32),
            pltpu.VMEM((b_per_w, D), _F32),
            pltpu.SemaphoreType.DMA,
        ],
    )
    def gather(feat_hbm, idx_hbm, out_hbm, idx_v, rows_v, sem):
        wid = lax.axis_index("s") * NC + lax.axis_index("c")
        pltpu.sync_copy(idx_hbm.at[wid], idx_v)
        cps = [
            pltpu.async_copy(
                feat_hbm.at[idx_v.at[j]], rows_v.at[pl.ds(j * CH, CH)], sem
            )
            for j in range(n_ch)
        ]
        for c in cps:
            c.wait()
        pltpu.sync_copy(rows_v, out_hbm.at[pl.ds(wid * b_per_w, b_per_w)])

    def run(features, idx3d):
        return gather(features, idx3d)

    return run, NW, n_ch, CH


@functools.lru_cache(maxsize=None)
def _make_sc_gather_l2():
    info = plsc.get_sparse_core_info()
    NC, NS = info.num_cores, info.num_subcores
    NW = NC * NS
    B_ = N1 + B
    b_per_w = B_ // NW
    mesh = plsc.VectorSubcoreMesh(core_axis_name="c", subcore_axis_name="s")

    @functools.partial(
        pl.kernel,
        mesh=mesh,
        out_type=jax.ShapeDtypeStruct((B_, HID), _F32),
        scratch_types=[
            pltpu.VMEM((1, b_per_w), jnp.int32),
            pltpu.VMEM((b_per_w, HID), _F32),
            pltpu.SemaphoreType.DMA,
        ],
    )
    def gather(t_hbm, idx_hbm, out_hbm, idx_v, rows_v, sem):
        wid = lax.axis_index("s") * NC + lax.axis_index("c")
        pltpu.sync_copy(idx_hbm.at[wid], idx_v)
        pltpu.async_copy(t_hbm.at[idx_v.at[0]], rows_v, sem).wait()
        pltpu.sync_copy(rows_v, out_hbm.at[pl.ds(wid * b_per_w, b_per_w)])

    def run(t, idx_all):
        return gather(t, idx_all.reshape(NW, 1, b_per_w))

    return run


def _l1_body(KB, m1t_ref, m1b_ref, src_ref, dst_ref, w_ref, av_ref, out_ref,
             acc_ref):
    k = pl.program_id(0)

    @pl.when(k == 0)
    def _():
        acc_ref[...] = jnp.zeros_like(acc_ref)

    s = src_ref[...]
    acc_ref[0 : N1 // 2, :] += jnp.dot(m1t_ref[...], s, preferred_element_type=_F32)
    acc_ref[N1 // 2 : N1, :] += jnp.dot(m1b_ref[...], s, preferred_element_type=_F32)

    @pl.when(k == KB - 1)
    def _():
        w = w_ref[...]
        h = jnp.dot(acc_ref[...], w[:D, :], preferred_element_type=_F32)
        h += jnp.dot(dst_ref[...], w[D:, :], preferred_element_type=_F32)
        logits = jnp.dot(h, av_ref[:HID, :], preferred_element_type=_F32)
        e = jnp.exp(logits - jnp.max(logits))
        out_ref[...] = (e / jnp.sum(e)) * h


def _l1_call(m1, rows, W1, av, src_blk0, dst_blk0, BK=1024):
    KB = N2 // BK
    return pl.pallas_call(
        functools.partial(_l1_body, KB),
        grid=(KB,),
        in_specs=[
            pl.BlockSpec((N1 // 2, BK), lambda k: (0, k)),
            pl.BlockSpec((N1 // 2, BK), lambda k: (1, k)),
            pl.BlockSpec((BK, D), lambda k: (src_blk0 + k, 0)),
            pl.BlockSpec((N1, D), lambda k: (dst_blk0, 0)),
            pl.BlockSpec((2 * D, HID), lambda k: (0, 0)),
            pl.BlockSpec((2 * HID, 1), lambda k: (0, 0)),
        ],
        out_specs=pl.BlockSpec((N1, HID), lambda k: (0, 0)),
        out_shape=jax.ShapeDtypeStruct((N1, HID), _F32),
        scratch_shapes=[pltpu.VMEM((N1, HID), _F32)],
    )(m1, m1, rows, rows, W1, av)


def _l2_body(KB2, g0s_ref, g0d_ref, g1s_ref, g1d_ref, m20_ref, m21_ref,
             w_ref, av_ref, wc_ref, out_ref, acc0_ref, acc1_ref):
    k = pl.program_id(0)

    @pl.when(k == 0)
    def _():
        acc0_ref[...] = jnp.zeros_like(acc0_ref)
        acc1_ref[...] = jnp.zeros_like(acc1_ref)

    acc0_ref[...] += jnp.dot(m20_ref[...], g0s_ref[...], preferred_element_type=_F32)
    acc1_ref[...] += jnp.dot(m21_ref[...], g1s_ref[...], preferred_element_type=_F32)

    @pl.when(k == KB2 - 1)
    def _():
        w = w_ref[...]
        av0 = av_ref[:HID, :]

        def rel(aggf, dst2):
            h = jnp.dot(aggf, w[:HID, :], preferred_element_type=_F32)
            h += jnp.dot(dst2, w[HID:, :], preferred_element_type=_F32)
            logits = jnp.dot(h, av0, preferred_element_type=_F32)
            e = jnp.exp(logits - jnp.max(logits))
            return (e / jnp.sum(e)) * h

        s = rel(acc0_ref[...], g0d_ref[...]) + rel(acc1_ref[...], g1d_ref[...])
        s *= lax.rsqrt(jnp.maximum(jnp.sum(s * s, axis=1, keepdims=True), 1e-12))
        p = jnp.dot(s, wc_ref[...], preferred_element_type=_F32)
        ep = jnp.exp(p - jnp.max(p, axis=1, keepdims=True))
        out_ref[...] = ep / jnp.sum(ep, axis=1, keepdims=True)


def _l2_call(g0, g1, m20, m21, W2, av, W_cls, BK2=512):
    C = W_cls.shape[1]
    KB2 = N1 // BK2
    return pl.pallas_call(
        functools.partial(_l2_body, KB2),
        grid=(KB2,),
        in_specs=[
            pl.BlockSpec((BK2, HID), lambda k: (k, 0)),
            pl.BlockSpec((B, HID), lambda k: (N1 // B, 0)),
            pl.BlockSpec((BK2, HID), lambda k: (k, 0)),
            pl.BlockSpec((B, HID), lambda k: (N1 // B, 0)),
            pl.BlockSpec((B, BK2), lambda k: (0, k)),
            pl.BlockSpec((B, BK2), lambda k: (0, k)),
            pl.BlockSpec((2 * HID, HID), lambda k: (0, 0)),
            pl.BlockSpec((2 * HID, 1), lambda k: (0, 0)),
            pl.BlockSpec((HID, C), lambda k: (0, 0)),
        ],
        out_specs=pl.BlockSpec((B, C), lambda k: (0, 0)),
        out_shape=jax.ShapeDtypeStruct((B, C), _F32),
        scratch_shapes=[pltpu.VMEM((B, HID), _F32), pltpu.VMEM((B, HID), _F32)],
    )(g0, g0, g1, g1, m20, m21, W2, av, W_cls)


def kernel(features, src_nodes_r0, dstsrc2src_l1_r0, dstsrc2dst_l1_r0,
           dif_mat_l1_r0, dstsrc2src_l2_r0, dstsrc2dst_l2_r0, dif_mat_l2_r0,
           src_nodes_r1, dstsrc2src_l1_r1, dstsrc2dst_l1_r1, dif_mat_l1_r1,
           dstsrc2src_l2_r1, dstsrc2dst_l2_r1, dif_mat_l2_r1,
           relation_vectors, attention_vec, W1, W2, W_cls):
    del relation_vectors

    gather_l1, NW, n_ch, CH = _make_sc_gather_l1()
    sn_cat = jnp.concatenate([src_nodes_r0, src_nodes_r1])
    s_cat = jnp.concatenate([
        dstsrc2src_l1_r0, dstsrc2dst_l1_r0,
        dstsrc2src_l1_r1 + N2, dstsrc2dst_l1_r1 + N2,
    ])
    idx_a = sn_cat.at[s_cat.reshape(NW, n_ch, CH)].get(
        mode="promise_in_bounds")
    rows = gather_l1(features, idx_a)

    x1_r0 = _l1_call(dif_mat_l1_r0, rows, W1, attention_vec, 0, 4)
    x1_r1 = _l1_call(dif_mat_l1_r1, rows, W1, attention_vec, 10, 9)

    g2 = _make_sc_gather_l2()
    g_r0 = g2(x1_r0, jnp.concatenate([dstsrc2src_l2_r0, dstsrc2dst_l2_r0]))
    g_r1 = g2(x1_r1, jnp.concatenate([dstsrc2src_l2_r1, dstsrc2dst_l2_r1]))

    return _l2_call(g_r0, g_r1, dif_mat_l2_r0, dif_mat_l2_r1,
                    W2, attention_vec, W_cls)


# speedup vs baseline: 1.0308x; 1.0308x over previous
import functools

import jax
import jax.numpy as jnp
from jax import lax
from jax.experimental import pallas as pl
from jax.experimental.pallas import tpu as pltpu
from jax.experimental.pallas import tpu_sc as plsc

N1 = 2048
N2 = 8192
B = 512
D = 128
HID = 128

_F32 = jnp.float32

_L = 16


@functools.lru_cache(maxsize=None)
def _make_sc_gather_l1():
    info = plsc.get_sparse_core_info()
    NC, NS = info.num_cores, info.num_subcores
    NW = NC * NS
    B_ = 2 * (N2 + N1)
    b_per_w = B_ // NW
    CH = 128
    n_ch = b_per_w // CH
    mesh = plsc.VectorSubcoreMesh(core_axis_name="c", subcore_axis_name="s")

    @functools.partial(
        pl.kernel,
        mesh=mesh,
        out_type=jax.ShapeDtypeStruct((B_, D), _F32),
        scratch_types=[
            pltpu.VMEM((n_ch, CH), jnp.int32),
            pltpu.VMEM((b_per_w, D), _F32),
            pltpu.SemaphoreType.DMA,
        ],
    )
    def gather(feat_hbm, idx_hbm, out_hbm, idx_v, rows_v, sem):
        wid = lax.axis_index("s") * NC + lax.axis_index("c")
        pltpu.sync_copy(idx_hbm.at[wid], idx_v)
        cps = [
            pltpu.async_copy(
                feat_hbm.at[idx_v.at[j]], rows_v.at[pl.ds(j * CH, CH)], sem
            )
            for j in range(n_ch)
        ]
        for c in cps:
            c.wait()
        pltpu.sync_copy(rows_v, out_hbm.at[pl.ds(wid * b_per_w, b_per_w)])

    def run(features, idx3d):
        return gather(features, idx3d)

    return run, NW, n_ch, CH


@functools.lru_cache(maxsize=None)
def _make_sc_gather_l2():
    info = plsc.get_sparse_core_info()
    NC, NS = info.num_cores, info.num_subcores
    NW = NC * NS
    B_ = N1 + B
    b_per_w = B_ // NW
    mesh = plsc.VectorSubcoreMesh(core_axis_name="c", subcore_axis_name="s")

    @functools.partial(
        pl.kernel,
        mesh=mesh,
        out_type=jax.ShapeDtypeStruct((B_, HID), _F32),
        scratch_types=[
            pltpu.VMEM((1, b_per_w), jnp.int32),
            pltpu.VMEM((b_per_w, HID), _F32),
            pltpu.SemaphoreType.DMA,
        ],
    )
    def gather(t_hbm, idx_hbm, out_hbm, idx_v, rows_v, sem):
        wid = lax.axis_index("s") * NC + lax.axis_index("c")
        pltpu.sync_copy(idx_hbm.at[wid], idx_v)
        pltpu.async_copy(t_hbm.at[idx_v.at[0]], rows_v, sem).wait()
        pltpu.sync_copy(rows_v, out_hbm.at[pl.ds(wid * b_per_w, b_per_w)])

    def run(t, idx_all):
        return gather(t, idx_all.reshape(NW, 1, b_per_w))

    return run


def _l1_body(KB, m1_ref, src_ref, dst_ref, w_ref, av_ref, out_ref, acc_ref):
    k = pl.program_id(0)

    @pl.when(k == 0)
    def _():
        acc_ref[...] = jnp.zeros_like(acc_ref)

    acc_ref[...] += jnp.dot(m1_ref[...], src_ref[...], preferred_element_type=_F32)

    @pl.when(k == KB - 1)
    def _():
        w = w_ref[...]
        h = jnp.dot(acc_ref[...], w[:D, :], preferred_element_type=_F32)
        h += jnp.dot(dst_ref[...], w[D:, :], preferred_element_type=_F32)
        logits = jnp.dot(h, av_ref[:HID, :], preferred_element_type=_F32)
        e = jnp.exp(logits - jnp.max(logits))
        out_ref[...] = (e / jnp.sum(e)) * h


def _l1_call(m1, rows, W1, av, src_blk0, dst_blk0, BK=1024):
    KB = N2 // BK
    return pl.pallas_call(
        functools.partial(_l1_body, KB),
        grid=(KB,),
        in_specs=[
            pl.BlockSpec((N1, BK), lambda k: (0, k)),
            pl.BlockSpec((BK, D), lambda k: (src_blk0 + k, 0)),
            pl.BlockSpec((N1, D), lambda k: (dst_blk0, 0)),
            pl.BlockSpec((2 * D, HID), lambda k: (0, 0)),
            pl.BlockSpec((2 * HID, 1), lambda k: (0, 0)),
        ],
        out_specs=pl.BlockSpec((N1, HID), lambda k: (0, 0)),
        out_shape=jax.ShapeDtypeStruct((N1, HID), _F32),
        scratch_shapes=[pltpu.VMEM((N1, HID), _F32)],
    )(m1, rows, rows, W1, av)


def _l2_body(KB2, g0s_ref, g0d_ref, g1s_ref, g1d_ref, m20_ref, m21_ref,
             w_ref, av_ref, wc_ref, out_ref, acc0_ref, acc1_ref):
    k = pl.program_id(0)

    @pl.when(k == 0)
    def _():
        acc0_ref[...] = jnp.zeros_like(acc0_ref)
        acc1_ref[...] = jnp.zeros_like(acc1_ref)

    acc0_ref[...] += jnp.dot(m20_ref[...], g0s_ref[...], preferred_element_type=_F32)
    acc1_ref[...] += jnp.dot(m21_ref[...], g1s_ref[...], preferred_element_type=_F32)

    @pl.when(k == KB2 - 1)
    def _():
        w = w_ref[...]
        av0 = av_ref[:HID, :]

        def rel(aggf, dst2):
            h = jnp.dot(aggf, w[:HID, :], preferred_element_type=_F32)
            h += jnp.dot(dst2, w[HID:, :], preferred_element_type=_F32)
            logits = jnp.dot(h, av0, preferred_element_type=_F32)
            e = jnp.exp(logits - jnp.max(logits))
            return (e / jnp.sum(e)) * h

        s = rel(acc0_ref[...], g0d_ref[...]) + rel(acc1_ref[...], g1d_ref[...])
        s *= lax.rsqrt(jnp.maximum(jnp.sum(s * s, axis=1, keepdims=True), 1e-12))
        p = jnp.dot(s, wc_ref[...], preferred_element_type=_F32)
        ep = jnp.exp(p - jnp.max(p, axis=1, keepdims=True))
        out_ref[...] = ep / jnp.sum(ep, axis=1, keepdims=True)


def _l2_call(g0, g1, m20, m21, W2, av, W_cls, BK2=512):
    C = W_cls.shape[1]
    KB2 = N1 // BK2
    return pl.pallas_call(
        functools.partial(_l2_body, KB2),
        grid=(KB2,),
        in_specs=[
            pl.BlockSpec((BK2, HID), lambda k: (k, 0)),
            pl.BlockSpec((B, HID), lambda k: (N1 // B, 0)),
            pl.BlockSpec((BK2, HID), lambda k: (k, 0)),
            pl.BlockSpec((B, HID), lambda k: (N1 // B, 0)),
            pl.BlockSpec((B, BK2), lambda k: (0, k)),
            pl.BlockSpec((B, BK2), lambda k: (0, k)),
            pl.BlockSpec((2 * HID, HID), lambda k: (0, 0)),
            pl.BlockSpec((2 * HID, 1), lambda k: (0, 0)),
            pl.BlockSpec((HID, C), lambda k: (0, 0)),
        ],
        out_specs=pl.BlockSpec((B, C), lambda k: (0, 0)),
        out_shape=jax.ShapeDtypeStruct((B, C), _F32),
        scratch_shapes=[pltpu.VMEM((B, HID), _F32), pltpu.VMEM((B, HID), _F32)],
    )(g0, g0, g1, g1, m20, m21, W2, av, W_cls)


def kernel(features, src_nodes_r0, dstsrc2src_l1_r0, dstsrc2dst_l1_r0,
           dif_mat_l1_r0, dstsrc2src_l2_r0, dstsrc2dst_l2_r0, dif_mat_l2_r0,
           src_nodes_r1, dstsrc2src_l1_r1, dstsrc2dst_l1_r1, dif_mat_l1_r1,
           dstsrc2src_l2_r1, dstsrc2dst_l2_r1, dif_mat_l2_r1,
           relation_vectors, attention_vec, W1, W2, W_cls):
    del relation_vectors

    gather_l1, NW, n_ch, CH = _make_sc_gather_l1()
    sn_cat = jnp.concatenate([src_nodes_r0, src_nodes_r1])
    s_cat = jnp.concatenate([
        dstsrc2src_l1_r0, dstsrc2dst_l1_r0,
        dstsrc2src_l1_r1 + N2, dstsrc2dst_l1_r1 + N2,
    ])
    idx_a = sn_cat.at[s_cat.reshape(NW, n_ch, CH)].get(
        mode="promise_in_bounds")
    rows = gather_l1(features, idx_a)

    x1_r0 = _l1_call(dif_mat_l1_r0, rows, W1, attention_vec, 0, 4)
    x1_r1 = _l1_call(dif_mat_l1_r1, rows, W1, attention_vec, 10, 9)

    g2 = _make_sc_gather_l2()
    g_r0 = g2(x1_r0, jnp.concatenate([dstsrc2src_l2_r0, dstsrc2dst_l2_r0]))
    g_r1 = g2(x1_r1, jnp.concatenate([dstsrc2src_l2_r1, dstsrc2dst_l2_r1]))

    return _l2_call(g_r0, g_r1, dif_mat_l2_r0, dif_mat_l2_r1,
                    W2, attention_vec, W_cls)
